# baseline (speedup 1.0000x reference)
import functools

import jax
import jax.numpy as jnp
from jax import lax
from jax.experimental import pallas as pl
from jax.experimental.pallas import tpu as pltpu
from jax.experimental.pallas import tpu_sc as plsc

N = 10000
E = 320000
F = 128
H = 4
C = 32
HC = H * C

NC = 2
NS = 16
NW = NC * NS
EW = E // NW
K = 80
NCH = EW // K
IB = 5
NOB = NCH // IB

_MESH = plsc.VectorSubcoreMesh(core_axis_name="c", subcore_axis_name="s")
_PREC = jax.lax.Precision.HIGHEST


def _proj_body(x_ref, wl_ref, wr_ref, xl_ref, xr_ref):
    xb = x_ref[...]
    xl_ref[...] = jnp.dot(xb, wl_ref[...], preferred_element_type=jnp.float32,
                          precision=_PREC)
    xr_ref[...] = jnp.dot(xb, wr_ref[...], preferred_element_type=jnp.float32,
                          precision=_PREC)


def _proj(x, W_l, W_r):
    blk = 2000
    return pl.pallas_call(
        _proj_body,
        grid=(N // blk,),
        in_specs=[
            pl.BlockSpec((blk, F), lambda i: (i, 0)),
            pl.BlockSpec((F, HC), lambda i: (0, 0)),
            pl.BlockSpec((F, HC), lambda i: (0, 0)),
        ],
        out_specs=[
            pl.BlockSpec((blk, HC), lambda i: (i, 0)),
            pl.BlockSpec((blk, HC), lambda i: (i, 0)),
        ],
        out_shape=[
            jax.ShapeDtypeStruct((N, HC), jnp.float32),
            jax.ShapeDtypeStruct((N, HC), jnp.float32),
        ],
    )(x, W_l, W_r)


def _sc_pass_a(xl, xr, src4, dst4, edge_attr, z128):
    @functools.partial(
        pl.kernel,
        out_type=[
            jax.ShapeDtypeStruct((E, HC), jnp.float32),
            jax.ShapeDtypeStruct((E, HC), jnp.float32),
            jax.ShapeDtypeStruct((NC, N, HC), jnp.float32),
        ],
        mesh=_MESH,
        scratch_types=[
            pltpu.VMEM((IB, K), jnp.int32),
            pltpu.VMEM((IB, K), jnp.int32),
            pltpu.VMEM((K, HC), jnp.float32),
            pltpu.VMEM((K, HC), jnp.float32),
            pltpu.VMEM((K, HC), jnp.float32),
            pltpu.VMEM_SHARED((N, HC), jnp.float32),
            pltpu.SemaphoreType.DMA,
            pltpu.SemaphoreType.DMA,
            pltpu.SemaphoreType.DMA,
            pltpu.SemaphoreType.DMA,
            pltpu.SemaphoreType.DMA,
            pltpu.SemaphoreType.DMA,
        ],
    )
    def k(xl_hbm, xr_hbm, src_hbm, dst_hbm, ea_hbm, z128_hbm,
          xs_hbm, xrd_hbm, sums_hbm,
          sidx, didx, xlb, xrb, eab, sums_sh,
          sA, sB, sC, sD, sE, sF):
        cid = lax.axis_index("c")
        sid = lax.axis_index("s")
        wid = sid * NC + cid
        base = wid * EW

        @pl.when(sid == 0)
        def _():
            pltpu.sync_copy(z128_hbm, sums_sh)

        plsc.subcore_barrier()

        @pl.loop(0, NOB)
        def _(ob):
            pltpu.sync_copy(src_hbm.at[wid, ob], sidx)
            pltpu.sync_copy(dst_hbm.at[wid, ob], didx)

            @pl.loop(0, IB)
            def _(jb):
                off = base + (ob * IB + jb) * K
                hA = pltpu.async_copy(xl_hbm.at[sidx.at[jb]], xlb, sA)
                hB = pltpu.async_copy(xr_hbm.at[didx.at[jb]], xrb, sB)
                hC = pltpu.async_copy(ea_hbm.at[pl.ds(off, K)], eab, sC)
                hA.wait()
                hD = pltpu.async_copy(xlb, xs_hbm.at[pl.ds(off, K)], sD)
                hB.wait()
                hE = pltpu.async_copy(xrb, xrd_hbm.at[pl.ds(off, K)], sE)
                hC.wait()
                hF = pltpu.async_copy(eab, sums_sh.at[didx.at[jb]], sF, add=True)
                hD.wait()
                hE.wait()
                hF.wait()

        plsc.subcore_barrier()

        @pl.when(sid == 0)
        def _():
            pltpu.sync_copy(sums_sh, sums_hbm.at[cid])

    return k(xl, xr, src4, dst4, edge_attr, z128)


def _attn_body(ea_ref, xs_ref, xrd_ref, we_ref, a_ref, b_ref, p128w_ref,
               c128_ref, p128_ref, scaled_ref):
    e = jnp.dot(ea_ref[...], we_ref[...], preferred_element_type=jnp.float32,
                precision=_PREC)
    xs = xs_ref[...]
    m = xs + xrd_ref[...] + e
    m = jnp.maximum(m, 0.2 * m)
    lg = jnp.dot(m, a_ref[...], preferred_element_type=jnp.float32,
                 precision=_PREC)
    p = jnp.exp(lg)
    p128_ref[...] = (
        jnp.dot(p, p128w_ref[...], preferred_element_type=jnp.float32)
        + c128_ref[...]
    )
    scaled_ref[...] = xs * jnp.dot(p, b_ref[...],
                                   preferred_element_type=jnp.float32)


def _attn(edge_attr, xs, xrd, W_e, A, B4, P128, c128):
    blk = 2000
    return pl.pallas_call(
        _attn_body,
        grid=(E // blk,),
        in_specs=[
            pl.BlockSpec((blk, F), lambda i: (i, 0)),
            pl.BlockSpec((blk, HC), lambda i: (i, 0)),
            pl.BlockSpec((blk, HC), lambda i: (i, 0)),
            pl.BlockSpec((F, HC), lambda i: (0, 0)),
            pl.BlockSpec((HC, H), lambda i: (0, 0)),
            pl.BlockSpec((H, HC), lambda i: (0, 0)),
            pl.BlockSpec((H, HC), lambda i: (0, 0)),
            pl.BlockSpec((1, HC), lambda i: (0, 0)),
        ],
        out_specs=[
            pl.BlockSpec((blk, HC), lambda i: (i, 0)),
            pl.BlockSpec((blk, HC), lambda i: (i, 0)),
        ],
        out_shape=[
            jax.ShapeDtypeStruct((E, HC), jnp.float32),
            jax.ShapeDtypeStruct((E, HC), jnp.float32),
        ],
    )(edge_attr, xs, xrd, W_e, A, B4, P128, c128)


def _sc_segsum(vals, dst4, z128):
    @functools.partial(
        pl.kernel,
        out_type=[
            jax.ShapeDtypeStruct((NC, N, HC), jnp.float32),
        ],
        mesh=_MESH,
        scratch_types=[
            pltpu.VMEM((IB, K), jnp.int32),
            pltpu.VMEM((K, HC), jnp.float32),
            pltpu.VMEM_SHARED((N, HC), jnp.float32),
        ],
    )
    def k(v_hbm, dst_hbm, z128_hbm, acc_hbm, didx, vb, acc_sh):
        cid = lax.axis_index("c")
        sid = lax.axis_index("s")
        wid = sid * NC + cid
        base = wid * EW

        @pl.when(sid == 0)
        def _():
            pltpu.sync_copy(z128_hbm, acc_sh)

        plsc.subcore_barrier()

        @pl.loop(0, NOB)
        def _(ob):
            pltpu.sync_copy(dst_hbm.at[wid, ob], didx)

            @pl.loop(0, IB)
            def _(jb):
                off = base + (ob * IB + jb) * K
                pltpu.sync_copy(v_hbm.at[pl.ds(off, K)], vb)
                pltpu.sync_copy(vb, acc_sh.at[didx.at[jb]], add=True)

        plsc.subcore_barrier()

        @pl.when(sid == 0)
        def _():
            pltpu.sync_copy(acc_sh, acc_hbm.at[cid])

    return k(vals, dst4, z128)[0]


def _final_body(s0_ref, s1_ref, a0_ref, a1_ref, d0_ref, d1_ref,
                xl_ref, xr_ref, we_ref, a_ref, b_ref, bias_ref, out_ref):
    xl = xl_ref[...]
    d128 = d0_ref[...] + d1_ref[...]
    sums = s0_ref[...] + s1_ref[...]
    cnt = jnp.maximum(d128[:, H:H + 1], 1.0)
    la = sums / cnt
    le = jnp.dot(la, we_ref[...], preferred_element_type=jnp.float32,
                 precision=_PREC)
    m = xl + xr_ref[...] + le
    m = jnp.maximum(m, 0.2 * m)
    lg = jnp.dot(m, a_ref[...], preferred_element_type=jnp.float32,
                 precision=_PREC)
    lp = jnp.exp(lg)
    dtot = d128[:, :H] + lp
    inv = 1.0 / (dtot + 1e-16)
    acc = a0_ref[...] + a1_ref[...]
    out = acc * jnp.dot(inv, b_ref[...], preferred_element_type=jnp.float32,
                        precision=_PREC)
    out = out + xl * jnp.dot(lp * inv, b_ref[...],
                             preferred_element_type=jnp.float32,
                             precision=_PREC)
    out_ref[...] = out + bias_ref[...]


def _final(sums_pc, acc_pc, den_pc, xl, xr, W_e, A, B4, bias2):
    blk = 2000
    return pl.pallas_call(
        _final_body,
        grid=(N // blk,),
        in_specs=[
            pl.BlockSpec((blk, HC), lambda i: (i, 0)),
            pl.BlockSpec((blk, HC), lambda i: (i, 0)),
            pl.BlockSpec((blk, HC), lambda i: (i, 0)),
            pl.BlockSpec((blk, HC), lambda i: (i, 0)),
            pl.BlockSpec((blk, HC), lambda i: (i, 0)),
            pl.BlockSpec((blk, HC), lambda i: (i, 0)),
            pl.BlockSpec((blk, HC), lambda i: (i, 0)),
            pl.BlockSpec((blk, HC), lambda i: (i, 0)),
            pl.BlockSpec((F, HC), lambda i: (0, 0)),
            pl.BlockSpec((HC, H), lambda i: (0, 0)),
            pl.BlockSpec((H, HC), lambda i: (0, 0)),
            pl.BlockSpec((1, HC), lambda i: (0, 0)),
        ],
        out_specs=pl.BlockSpec((blk, HC), lambda i: (i, 0)),
        out_shape=jax.ShapeDtypeStruct((N, HC), jnp.float32),
    )(sums_pc[0], sums_pc[1], acc_pc[0], acc_pc[1], den_pc[0], den_pc[1],
      xl, xr, W_e, A, B4, bias2)


@jax.jit
def kernel(x, edge_index, edge_attr, W_l, W_r, W_e, att, bias):
    src4 = edge_index[0].reshape(NW, NOB, IB, K)
    dst4 = edge_index[1].reshape(NW, NOB, IB, K)
    z128 = jnp.zeros((N, HC), jnp.float32)
    attf = att.reshape(HC)
    B4 = jnp.repeat(jnp.eye(H, dtype=jnp.float32), C, axis=1)
    A = B4.T * attf[:, None]
    P128 = jnp.concatenate(
        [jnp.eye(H, dtype=jnp.float32), jnp.zeros((H, HC - H), jnp.float32)],
        axis=1)
    c128 = jnp.zeros((1, HC), jnp.float32).at[0, H].set(1.0)
    bias2 = bias.reshape(1, HC)

    xl, xr = _proj(x, W_l, W_r)
    xs, xrd, sums_pc = _sc_pass_a(xl, xr, src4, dst4, edge_attr, z128)
    p128, scaled = _attn(edge_attr, xs, xrd, W_e, A, B4, P128, c128)
    acc_pc = _sc_segsum(scaled, dst4, z128)
    den_pc = _sc_segsum(p128, dst4, z128)
    return _final(sums_pc, acc_pc, den_pc, xl, xr, W_e, A, B4, bias2)


# speedup vs baseline: 1.2316x; 1.2316x over previous
import functools

import jax
import jax.numpy as jnp
from jax import lax
from jax.experimental import pallas as pl
from jax.experimental.pallas import tpu as pltpu
from jax.experimental.pallas import tpu_sc as plsc

N = 10000
E = 320000
F = 128
H = 4
C = 32
HC = H * C

NC = 2
NS = 16
NW = NC * NS
EW = E // NW
K = 80
NCH = EW // K
IB = 5
NOB = NCH // IB

_MESH = plsc.VectorSubcoreMesh(core_axis_name="c", subcore_axis_name="s")
_PREC = jax.lax.Precision.HIGHEST


def _proj_body(x_ref, wl_ref, wr_ref, xl_ref, xr_ref):
    xb = x_ref[...]
    xl_ref[...] = jnp.dot(xb, wl_ref[...], preferred_element_type=jnp.float32,
                          precision=_PREC)
    xr_ref[...] = jnp.dot(xb, wr_ref[...], preferred_element_type=jnp.float32,
                          precision=_PREC)


def _proj(x, W_l, W_r):
    blk = 2000
    return pl.pallas_call(
        _proj_body,
        grid=(N // blk,),
        in_specs=[
            pl.BlockSpec((blk, F), lambda i: (i, 0)),
            pl.BlockSpec((F, HC), lambda i: (0, 0)),
            pl.BlockSpec((F, HC), lambda i: (0, 0)),
        ],
        out_specs=[
            pl.BlockSpec((blk, HC), lambda i: (i, 0)),
            pl.BlockSpec((blk, HC), lambda i: (i, 0)),
        ],
        out_shape=[
            jax.ShapeDtypeStruct((N, HC), jnp.float32),
            jax.ShapeDtypeStruct((N, HC), jnp.float32),
        ],
    )(x, W_l, W_r)


def _sc_pass_a(xl, xr, src4, dst4, edge_attr, z128):
    @functools.partial(
        pl.kernel,
        out_type=[
            jax.ShapeDtypeStruct((E, HC), jnp.float32),
            jax.ShapeDtypeStruct((E, HC), jnp.float32),
            jax.ShapeDtypeStruct((NC, N, HC), jnp.float32),
        ],
        mesh=_MESH,
        scratch_types=[
            pltpu.VMEM((IB, K), jnp.int32),
            pltpu.VMEM((IB, K), jnp.int32),
            pltpu.VMEM((K, HC), jnp.float32),
            pltpu.VMEM((K, HC), jnp.float32),
            pltpu.VMEM((K, HC), jnp.float32),
            pltpu.VMEM_SHARED((N, HC), jnp.float32),
            pltpu.SemaphoreType.DMA,
            pltpu.SemaphoreType.DMA,
            pltpu.SemaphoreType.DMA,
            pltpu.SemaphoreType.DMA,
            pltpu.SemaphoreType.DMA,
            pltpu.SemaphoreType.DMA,
        ],
    )
    def k(xl_hbm, xr_hbm, src_hbm, dst_hbm, ea_hbm, z128_hbm,
          xs_hbm, xrd_hbm, sums_hbm,
          sidx, didx, xlb, xrb, eab, sums_sh,
          sA, sB, sC, sD, sE, sF):
        cid = lax.axis_index("c")
        sid = lax.axis_index("s")
        wid = sid * NC + cid
        base = wid * EW

        @pl.when(sid == 0)
        def _():
            pltpu.sync_copy(z128_hbm, sums_sh)

        plsc.subcore_barrier()

        @pl.loop(0, NOB)
        def _(ob):
            pltpu.sync_copy(src_hbm.at[wid, ob], sidx)
            pltpu.sync_copy(dst_hbm.at[wid, ob], didx)

            @pl.loop(0, IB)
            def _(jb):
                off = base + (ob * IB + jb) * K
                hA = pltpu.async_copy(xl_hbm.at[sidx.at[jb]], xlb, sA)
                hB = pltpu.async_copy(xr_hbm.at[didx.at[jb]], xrb, sB)
                hC = pltpu.async_copy(ea_hbm.at[pl.ds(off, K)], eab, sC)
                hA.wait()
                hD = pltpu.async_copy(xlb, xs_hbm.at[pl.ds(off, K)], sD)
                hB.wait()
                hE = pltpu.async_copy(xrb, xrd_hbm.at[pl.ds(off, K)], sE)
                hC.wait()
                hF = pltpu.async_copy(eab, sums_sh.at[didx.at[jb]], sF, add=True)
                hD.wait()
                hE.wait()
                hF.wait()

        plsc.subcore_barrier()

        @pl.when(sid == 0)
        def _():
            pltpu.sync_copy(sums_sh, sums_hbm.at[cid])

    return k(xl, xr, src4, dst4, edge_attr, z128)


def _attn_body(ea_ref, xs_ref, xrd_ref, we_ref, a_ref, b_ref, p128w_ref,
               c128_ref, p128_ref, scaled_ref):
    e = jnp.dot(ea_ref[...], we_ref[...], preferred_element_type=jnp.float32)
    xs = xs_ref[...]
    m = xs + xrd_ref[...] + e
    m = jnp.maximum(m, 0.2 * m)
    lg = jnp.dot(m, a_ref[...], preferred_element_type=jnp.float32)
    p = jnp.exp(lg)
    p128_ref[...] = (
        jnp.dot(p, p128w_ref[...], preferred_element_type=jnp.float32)
        + c128_ref[...]
    )
    scaled_ref[...] = xs * jnp.dot(p, b_ref[...],
                                   preferred_element_type=jnp.float32)


def _attn(edge_attr, xs, xrd, W_e, A, B4, P128, c128):
    blk = 2000
    return pl.pallas_call(
        _attn_body,
        grid=(E // blk,),
        in_specs=[
            pl.BlockSpec((blk, F), lambda i: (i, 0)),
            pl.BlockSpec((blk, HC), lambda i: (i, 0)),
            pl.BlockSpec((blk, HC), lambda i: (i, 0)),
            pl.BlockSpec((F, HC), lambda i: (0, 0)),
            pl.BlockSpec((HC, H), lambda i: (0, 0)),
            pl.BlockSpec((H, HC), lambda i: (0, 0)),
            pl.BlockSpec((H, HC), lambda i: (0, 0)),
            pl.BlockSpec((1, HC), lambda i: (0, 0)),
        ],
        out_specs=[
            pl.BlockSpec((blk, HC), lambda i: (i, 0)),
            pl.BlockSpec((blk, HC), lambda i: (i, 0)),
        ],
        out_shape=[
            jax.ShapeDtypeStruct((E, HC), jnp.float32),
            jax.ShapeDtypeStruct((E, HC), jnp.float32),
        ],
    )(edge_attr, xs, xrd, W_e, A, B4, P128, c128)


def _sc_segsum(vals, dst4, z128):
    @functools.partial(
        pl.kernel,
        out_type=[
            jax.ShapeDtypeStruct((NC, N, HC), jnp.float32),
        ],
        mesh=_MESH,
        scratch_types=[
            pltpu.VMEM((IB, K), jnp.int32),
            pltpu.VMEM((K, HC), jnp.float32),
            pltpu.VMEM_SHARED((N, HC), jnp.float32),
        ],
    )
    def k(v_hbm, dst_hbm, z128_hbm, acc_hbm, didx, vb, acc_sh):
        cid = lax.axis_index("c")
        sid = lax.axis_index("s")
        wid = sid * NC + cid
        base = wid * EW

        @pl.when(sid == 0)
        def _():
            pltpu.sync_copy(z128_hbm, acc_sh)

        plsc.subcore_barrier()

        @pl.loop(0, NOB)
        def _(ob):
            pltpu.sync_copy(dst_hbm.at[wid, ob], didx)

            @pl.loop(0, IB)
            def _(jb):
                off = base + (ob * IB + jb) * K
                pltpu.sync_copy(v_hbm.at[pl.ds(off, K)], vb)
                pltpu.sync_copy(vb, acc_sh.at[didx.at[jb]], add=True)

        plsc.subcore_barrier()

        @pl.when(sid == 0)
        def _():
            pltpu.sync_copy(acc_sh, acc_hbm.at[cid])

    return k(vals, dst4, z128)[0]


def _final_body(s0_ref, s1_ref, a0_ref, a1_ref, d0_ref, d1_ref,
                xl_ref, xr_ref, we_ref, a_ref, b_ref, bias_ref, out_ref):
    xl = xl_ref[...]
    d128 = d0_ref[...] + d1_ref[...]
    sums = s0_ref[...] + s1_ref[...]
    cnt = jnp.maximum(d128[:, H:H + 1], 1.0)
    la = sums / cnt
    le = jnp.dot(la, we_ref[...], preferred_element_type=jnp.float32,
                 precision=_PREC)
    m = xl + xr_ref[...] + le
    m = jnp.maximum(m, 0.2 * m)
    lg = jnp.dot(m, a_ref[...], preferred_element_type=jnp.float32,
                 precision=_PREC)
    lp = jnp.exp(lg)
    dtot = d128[:, :H] + lp
    inv = 1.0 / (dtot + 1e-16)
    acc = a0_ref[...] + a1_ref[...]
    out = acc * jnp.dot(inv, b_ref[...], preferred_element_type=jnp.float32,
                        precision=_PREC)
    out = out + xl * jnp.dot(lp * inv, b_ref[...],
                             preferred_element_type=jnp.float32,
                             precision=_PREC)
    out_ref[...] = out + bias_ref[...]


def _final(sums_pc, acc_pc, den_pc, xl, xr, W_e, A, B4, bias2):
    blk = 2000
    return pl.pallas_call(
        _final_body,
        grid=(N // blk,),
        in_specs=[
            pl.BlockSpec((blk, HC), lambda i: (i, 0)),
            pl.BlockSpec((blk, HC), lambda i: (i, 0)),
            pl.BlockSpec((blk, HC), lambda i: (i, 0)),
            pl.BlockSpec((blk, HC), lambda i: (i, 0)),
            pl.BlockSpec((blk, HC), lambda i: (i, 0)),
            pl.BlockSpec((blk, HC), lambda i: (i, 0)),
            pl.BlockSpec((blk, HC), lambda i: (i, 0)),
            pl.BlockSpec((blk, HC), lambda i: (i, 0)),
            pl.BlockSpec((F, HC), lambda i: (0, 0)),
            pl.BlockSpec((HC, H), lambda i: (0, 0)),
            pl.BlockSpec((H, HC), lambda i: (0, 0)),
            pl.BlockSpec((1, HC), lambda i: (0, 0)),
        ],
        out_specs=pl.BlockSpec((blk, HC), lambda i: (i, 0)),
        out_shape=jax.ShapeDtypeStruct((N, HC), jnp.float32),
    )(sums_pc[0], sums_pc[1], acc_pc[0], acc_pc[1], den_pc[0], den_pc[1],
      xl, xr, W_e, A, B4, bias2)


@jax.jit
def kernel(x, edge_index, edge_attr, W_l, W_r, W_e, att, bias):
    src4 = edge_index[0].reshape(NW, NOB, IB, K)
    dst4 = edge_index[1].reshape(NW, NOB, IB, K)
    z128 = jnp.zeros((N, HC), jnp.float32)
    attf = att.reshape(HC)
    B4 = jnp.repeat(jnp.eye(H, dtype=jnp.float32), C, axis=1)
    A = B4.T * attf[:, None]
    P128 = jnp.concatenate(
        [jnp.eye(H, dtype=jnp.float32), jnp.zeros((H, HC - H), jnp.float32)],
        axis=1)
    c128 = jnp.zeros((1, HC), jnp.float32).at[0, H].set(1.0)
    bias2 = bias.reshape(1, HC)

    xl, xr = _proj(x, W_l, W_r)
    xs, xrd, sums_pc = _sc_pass_a(xl, xr, src4, dst4, edge_attr, z128)
    p128, scaled = _attn(edge_attr, xs, xrd, W_e, A, B4, P128, c128)
    acc_pc = _sc_segsum(scaled, dst4, z128)
    den_pc = _sc_segsum(p128, dst4, z128)
    return _final(sums_pc, acc_pc, den_pc, xl, xr, W_e, A, B4, bias2)
